# Initial kernel scaffold; baseline (speedup 1.0000x reference)
import jax
import jax.numpy as jnp
from jax.experimental import pallas as pl


def kernel(x, W1, b1, W2, b2):
    raise NotImplementedError("write your pallas kernel here")


# speedup vs baseline: 1.2169x; 1.2169x over previous
import functools

import jax
import jax.numpy as jnp
from jax.experimental import pallas as pl
from jax.experimental.pallas import tpu as pltpu

B, E, N, D, DFF = 2, 8, 512, 1024, 4096
BF = 1024
NJ = DFF // BF


def _ffn_kernel(x_ref, w1_ref, b1_ref, w2_ref, b2_ref, out_ref):
    j = pl.program_id(1)

    xt = x_ref[...].reshape(B * N, D).astype(jnp.bfloat16)
    w1 = w1_ref[...].reshape(D, BF).astype(jnp.bfloat16)
    h = jax.lax.dot_general(
        xt, w1, (((1,), (0,)), ((), ())), preferred_element_type=jnp.float32
    )
    h = jax.nn.gelu(h + b1_ref[...].reshape(1, BF))
    w2 = w2_ref[...].reshape(BF, D).astype(jnp.bfloat16)
    acc = jax.lax.dot_general(
        h.astype(jnp.bfloat16), w2, (((1,), (0,)), ((), ())),
        preferred_element_type=jnp.float32,
    )

    @pl.when(j == 0)
    def _():
        out_ref[...] = (acc + b2_ref[...].reshape(1, D)).reshape(B, 1, N, D)

    @pl.when(j > 0)
    def _():
        out_ref[...] += acc.reshape(B, 1, N, D)


@jax.jit
def kernel(x, W1, b1, W2, b2):
    grid = (E, NJ)
    out = pl.pallas_call(
        _ffn_kernel,
        grid=grid,
        in_specs=[
            pl.BlockSpec((B, 1, N, D), lambda e, j: (0, e, 0, 0)),
            pl.BlockSpec((1, D, BF), lambda e, j: (e, 0, j)),
            pl.BlockSpec((1, 1, BF), lambda e, j: (e, 0, j)),
            pl.BlockSpec((1, BF, D), lambda e, j: (e, j, 0)),
            pl.BlockSpec((1, 1, D), lambda e, j: (e, 0, 0)),
        ],
        out_specs=pl.BlockSpec((B, 1, N, D), lambda e, j: (0, e, 0, 0)),
        out_shape=jax.ShapeDtypeStruct((B, E, N, D), jnp.float32),
        compiler_params=pltpu.CompilerParams(
            dimension_semantics=("arbitrary", "arbitrary"),
        ),
    )(x, W1, b1.reshape(E, 1, DFF), W2, b2.reshape(E, 1, D))
    return out


# speedup vs baseline: 1.2820x; 1.0535x over previous
import functools

import jax
import jax.numpy as jnp
from jax.experimental import pallas as pl
from jax.experimental.pallas import tpu as pltpu

B, E, N, D, DFF = 2, 8, 512, 1024, 4096
BF = 1024
NJ = DFF // BF


def _ffn_kernel(x_ref, w1_ref, b1_ref, w2_ref, b2_ref, out_ref):
    j = pl.program_id(1)

    xt = x_ref[...].reshape(B * N, D).astype(jnp.bfloat16)
    w1 = w1_ref[...].reshape(D, BF).astype(jnp.bfloat16)
    h = jax.lax.dot_general(
        xt, w1, (((1,), (0,)), ((), ())), preferred_element_type=jnp.float32
    )
    h = jax.nn.gelu((h + b1_ref[...].reshape(1, BF)).astype(jnp.bfloat16))
    w2 = w2_ref[...].reshape(BF, D).astype(jnp.bfloat16)
    acc = jax.lax.dot_general(
        h, w2, (((1,), (0,)), ((), ())),
        preferred_element_type=jnp.float32,
    )

    @pl.when(j == 0)
    def _():
        out_ref[...] = (acc + b2_ref[...].reshape(1, D)).reshape(B, 1, N, D)

    @pl.when(j > 0)
    def _():
        out_ref[...] += acc.reshape(B, 1, N, D)


@jax.jit
def kernel(x, W1, b1, W2, b2):
    grid = (E, NJ)
    out = pl.pallas_call(
        _ffn_kernel,
        grid=grid,
        in_specs=[
            pl.BlockSpec((B, 1, N, D), lambda e, j: (0, e, 0, 0)),
            pl.BlockSpec((1, D, BF), lambda e, j: (e, 0, j)),
            pl.BlockSpec((1, 1, BF), lambda e, j: (e, 0, j)),
            pl.BlockSpec((1, BF, D), lambda e, j: (e, j, 0)),
            pl.BlockSpec((1, 1, D), lambda e, j: (e, 0, 0)),
        ],
        out_specs=pl.BlockSpec((B, 1, N, D), lambda e, j: (0, e, 0, 0)),
        out_shape=jax.ShapeDtypeStruct((B, E, N, D), jnp.float32),
        compiler_params=pltpu.CompilerParams(
            dimension_semantics=("arbitrary", "arbitrary"),
        ),
    )(x, W1, b1.reshape(E, 1, DFF), W2, b2.reshape(E, 1, D))
    return out


# speedup vs baseline: 1.3576x; 1.0590x over previous
import functools

import jax
import jax.numpy as jnp
from jax.experimental import pallas as pl
from jax.experimental.pallas import tpu as pltpu

B, E, N, D, DFF = 2, 8, 512, 1024, 4096
BF = 2048
NJ = DFF // BF


def _ffn_kernel(x_ref, w1_ref, b1_ref, w2_ref, b2_ref, out_ref):
    j = pl.program_id(1)

    xt = x_ref[...].reshape(B * N, D).astype(jnp.bfloat16)
    w1 = w1_ref[...].reshape(D, BF).astype(jnp.bfloat16)
    h = jax.lax.dot_general(
        xt, w1, (((1,), (0,)), ((), ())), preferred_element_type=jnp.float32
    )
    h = jax.nn.gelu((h + b1_ref[...].reshape(1, BF)).astype(jnp.bfloat16))
    w2 = w2_ref[...].reshape(BF, D).astype(jnp.bfloat16)
    acc = jax.lax.dot_general(
        h, w2, (((1,), (0,)), ((), ())),
        preferred_element_type=jnp.float32,
    )

    @pl.when(j == 0)
    def _():
        out_ref[...] = (acc + b2_ref[...].reshape(1, D)).reshape(B, 1, N, D)

    @pl.when(j > 0)
    def _():
        out_ref[...] += acc.reshape(B, 1, N, D)


@jax.jit
def kernel(x, W1, b1, W2, b2):
    grid = (E, NJ)
    out = pl.pallas_call(
        _ffn_kernel,
        grid=grid,
        in_specs=[
            pl.BlockSpec((B, 1, N, D), lambda e, j: (0, e, 0, 0)),
            pl.BlockSpec((1, D, BF), lambda e, j: (e, 0, j)),
            pl.BlockSpec((1, 1, BF), lambda e, j: (e, 0, j)),
            pl.BlockSpec((1, BF, D), lambda e, j: (e, j, 0)),
            pl.BlockSpec((1, 1, D), lambda e, j: (e, 0, 0)),
        ],
        out_specs=pl.BlockSpec((B, 1, N, D), lambda e, j: (0, e, 0, 0)),
        out_shape=jax.ShapeDtypeStruct((B, E, N, D), jnp.float32),
        compiler_params=pltpu.CompilerParams(
            dimension_semantics=("arbitrary", "arbitrary"),
        ),
    )(x, W1, b1.reshape(E, 1, DFF), W2, b2.reshape(E, 1, D))
    return out
